# baseline (speedup 1.0000x reference)
import functools

import jax
import jax.numpy as jnp
from jax import lax
from jax.experimental import pallas as pl
from jax.experimental.pallas import tpu as pltpu
from jax.experimental.pallas import tpu_sc as plsc

_N = 1024
_E = 16384
_HID = 256
_HALF = _HID // 2
_MAX_DEGREE = 5
_T = 1.0

_NS = 16
_LANES = 16
_EPT = _E // _NS
_CH = 128
_NCHUNK = _EPT // _CH


_ZBUF = 4096


def _adj_scatter_body(edge_hbm, out_hbm,
                      srcv, dstv, idxv, onesv, halfv, zbuf, shared, sem,
                      scat_sem):
    c = lax.axis_index("c")
    s = lax.axis_index("s")

    zero16 = jnp.zeros((_LANES,), jnp.float32)

    def _zfill(i, carry):
        zbuf[pl.ds(i * _LANES, _LANES)] = zero16
        return carry

    lax.fori_loop(0, _ZBUF // _LANES, _zfill, 0)
    zchunk = (_N * _N) // _NS
    zcopies = [
        pltpu.async_copy(zbuf, shared.at[pl.ds(s * zchunk + m * _ZBUF, _ZBUF)],
                         sem)
        for m in range(zchunk // _ZBUF)
    ]

    base = s * _EPT
    pltpu.sync_copy(edge_hbm.at[0, pl.ds(base, _EPT)], srcv)
    pltpu.sync_copy(edge_hbm.at[1, pl.ds(base, _EPT)], dstv)

    ones16 = jnp.full((_LANES,), 1.0, jnp.float32)
    zero16f = jnp.zeros((_LANES,), jnp.float32)
    for i in range(_CH // _LANES):
        onesv[pl.ds(i * _LANES, _LANES)] = ones16
        halfv[pl.ds(i * _LANES, _LANES)] = (
            ones16 if i < (_CH // _LANES) // 2 else zero16f)

    per_row = _CH // _LANES

    def _idx_fill(i, carry):
        sv = srcv[pl.ds(i * _LANES, _LANES)]
        dv = dstv[pl.ds(i * _LANES, _LANES)]
        row = jnp.where(c == 0, dv, sv)
        col = jnp.where(c == 0, sv, dv)
        idxv[i // per_row, pl.ds((i % per_row) * _LANES, _LANES)] = row * _N + col
        return carry

    lax.fori_loop(0, _EPT // _LANES, _idx_fill, 0)

    iota16 = lax.broadcasted_iota(jnp.int32, (_LANES,), 0)
    for i in range(_CH // _LANES):
        r16 = s * 64 + (i % 4) * _LANES + iota16
        idxv[_NCHUNK, pl.ds(i * _LANES, _LANES)] = r16 * (_N + 1)

    for zc in zcopies:
        zc.wait()
    plsc.subcore_barrier()
    descs = [pltpu.async_copy(onesv, shared.at[idxv.at[j]], scat_sem, add=True)
             for j in range(_NCHUNK)]

    @pl.when(c == 1)
    def _loops():
        pltpu.sync_copy(halfv, shared.at[idxv.at[_NCHUNK]], add=True)

    for d in descs:
        d.wait()
    plsc.subcore_barrier()
    pltpu.sync_copy(shared.at[pl.ds(s * zchunk, zchunk)],
                    out_hbm.at[c, pl.ds(s * zchunk, zchunk)])


@functools.cache
def _adj_scatter():
    return pl.kernel(
        _adj_scatter_body,
        out_type=jax.ShapeDtypeStruct((2, _N * _N), jnp.float32),
        mesh=plsc.VectorSubcoreMesh(core_axis_name="c", subcore_axis_name="s"),
        scratch_types=[
            pltpu.VMEM((_EPT,), jnp.int32),
            pltpu.VMEM((_EPT,), jnp.int32),
            pltpu.VMEM((_NCHUNK + 1, _CH), jnp.int32),
            pltpu.VMEM((_CH,), jnp.float32),
            pltpu.VMEM((_CH,), jnp.float32),
            pltpu.VMEM((_ZBUF,), jnp.float32),
            pltpu.VMEM_SHARED((_N * _N,), jnp.float32),
            pltpu.SemaphoreType.DMA,
            pltpu.SemaphoreType.DMA,
        ],
    )


def _rot_partner(z):
    zm1 = jnp.concatenate([z[:, 1:], z[:, :1]], axis=1)
    zp1 = jnp.concatenate([z[:, -1:], z[:, :-1]], axis=1)
    lane = lax.broadcasted_iota(jnp.int32, z.shape, 1)
    return jnp.where(lane % 2 == 0, zm1, -zp1)


def _dense_body(adj2_ref, x_ref, wl1_ref, wr1_ref, b1_ref, wl2_ref, wr2_ref,
                b2_ref, wa_ref, ba_ref, w_ref, bb_ref, out_ref):
    f32 = jnp.float32
    b1 = b1_ref[...].reshape(1, -1)
    b2 = b2_ref[...].reshape(1, -1)
    ba = ba_ref[...].reshape(1, 1)
    bb = bb_ref[...].reshape(1, -1)
    adjc = adj2_ref[0].reshape(_N, _N)
    adjs = adj2_ref[1].reshape(_N, _N)
    x = x_ref[...]

    cnt = jnp.sum(adjc, axis=1, keepdims=True)
    inv_cnt = 1.0 / jnp.maximum(cnt, 1.0)

    deg_col = jnp.sum(adjs, axis=1, keepdims=True)
    deg_row = 1.0 + jnp.sum(adjc, axis=0, keepdims=True)
    na = jnp.where(adjs > 0.0, lax.rsqrt(deg_col) * lax.rsqrt(deg_row), 0.0)

    bf16 = jnp.bfloat16
    adjc_b = adjc.astype(bf16)
    x_b = x.astype(bf16)
    mean1 = jnp.dot(adjc_b, x_b, preferred_element_type=f32) * inv_cnt
    h1 = jnp.maximum(
        jnp.dot(mean1.astype(bf16), wl1_ref[...].astype(bf16),
                preferred_element_type=f32)
        + jnp.dot(x_b, wr1_ref[...].astype(bf16), preferred_element_type=f32)
        + b1, 0.0)
    mean2 = jnp.dot(adjc_b, h1.astype(bf16), preferred_element_type=f32) * inv_cnt
    h2 = jnp.maximum(
        jnp.dot(mean2.astype(bf16), wl2_ref[...].astype(bf16),
                preferred_element_type=f32)
        + jnp.dot(h1.astype(bf16), wr2_ref[...].astype(bf16),
                  preferred_element_type=f32)
        + b2, 0.0)
    ang = jnp.dot(h2, wa_ref[...], preferred_element_type=f32) + ba
    co = jnp.cos(ang)
    si = jnp.sin(ang)

    z = jnp.dot(x_b, w_ref[...].astype(bf16), preferred_element_type=f32) + bb
    acc = co * z + si * _rot_partner(z)
    term = acc
    na_b = na.astype(bf16)
    for k in range(1, _MAX_DEGREE + 1):
        term = (-_T / k) * (
            term - jnp.dot(na_b, term.astype(bf16), preferred_element_type=f32))
        acc = acc + term

    out_ref[...] = jnp.maximum(co * acc - si * _rot_partner(acc), 0.0)


def kernel(x, edge_index, Wl1, Wr1, b1, Wl2, Wr2, b2, Wa, ba, W, b):
    adj2 = _adj_scatter()(edge_index)

    return pl.pallas_call(
        _dense_body,
        out_shape=jax.ShapeDtypeStruct((_N, _HID), jnp.float32),
    )(adj2, x, Wl1, Wr1, b1, Wl2, Wr2, b2, Wa, ba, W, b)
